# Initial kernel scaffold; baseline (speedup 1.0000x reference)
import jax
import jax.numpy as jnp
from jax.experimental import pallas as pl


def kernel(x, edge_index, edge_attr, batch, node_emb, edge_emb, msg_W1, msg_b1, msg_W2, msg_b2, gru_Wih, gru_Whh, gru_bih, gru_bhh, ri_W1, ri_b1, ri_W2, ri_b2, rj_W1, rj_b1, rj_W2, rj_b2):
    raise NotImplementedError("write your pallas kernel here")


# speedup vs baseline: 9.2043x; 9.2043x over previous
import functools

import jax
import jax.numpy as jnp
from jax import lax
from jax.experimental import pallas as pl
from jax.experimental.pallas import tpu as pltpu
from jax.experimental.pallas import tpu_sc as plsc

N = 10000
E = 640000
H = 128
NODE_VOCAB = 128
EDGE_VOCAB = 16
NUM_GRAPHS = 256
NCOMBO = NODE_VOCAB * EDGE_VOCAB

NPAD = 10240
BLK = 1024
NBLK = NPAD // BLK

NC = 2
NS = 16
NW = NC * NS
EPW = E // NW
CH = 80
NCHUNK = EPW // CH
ZROWS = NPAD // NS


def _table_body(ne, ee, w1, b1, w2, b2, out):
    pre_n = jnp.dot(ne[...], w1[:H, :], preferred_element_type=jnp.float32)
    pre_e = jnp.dot(ee[...], w1[H:, :], preferred_element_type=jnp.float32) + b1[...]
    row = lax.broadcasted_iota(jnp.int32, (NCOMBO, 1), 0)
    oa = (row // EDGE_VOCAB == lax.broadcasted_iota(jnp.int32, (NCOMBO, NODE_VOCAB), 1)).astype(jnp.float32)
    ob = (row % EDGE_VOCAB == lax.broadcasted_iota(jnp.int32, (NCOMBO, EDGE_VOCAB), 1)).astype(jnp.float32)
    z = jnp.dot(oa, pre_n, preferred_element_type=jnp.float32)
    z = z + jnp.dot(ob, pre_e, preferred_element_type=jnp.float32)
    out[...] = jnp.dot(jnp.maximum(z, 0.0), w2[...], preferred_element_type=jnp.float32) + b2[...]


def _make_table(interpret=False):
    return pl.pallas_call(
        _table_body,
        out_shape=jax.ShapeDtypeStruct((NCOMBO, H), jnp.float32),
        interpret=interpret,
    )


def _edge_body(src_hbm, dst_hbm, attr_hbm, x_hbm, mtab_hbm, zeros_hbm, out_hbm,
               x_v, src_v, dst_v, attr_v, combo_v, rows_v, aggr_sh, sem):
    c = lax.axis_index("c")
    s = lax.axis_index("s")

    pltpu.sync_copy(zeros_hbm, aggr_sh.at[pl.ds(s * ZROWS, ZROWS)])
    pltpu.sync_copy(x_hbm, x_v)
    plsc.subcore_barrier()

    base = (c * NS + s) * EPW

    def chunk(i, carry):
        off = base + i * CH
        pltpu.sync_copy(src_hbm.at[pl.ds(off, CH)], src_v)
        pltpu.sync_copy(attr_hbm.at[pl.ds(off, CH)], attr_v)
        pltpu.sync_copy(dst_hbm.at[pl.ds(off, CH)], dst_v)

        def cb(j, carry2):
            sl = pl.ds(j * 16, 16)
            xg = plsc.load_gather(x_v, [src_v[sl]])
            combo_v[sl] = xg * EDGE_VOCAB + attr_v[sl]
            return carry2

        lax.fori_loop(0, CH // 16, cb, 0, unroll=True)
        pltpu.async_copy(mtab_hbm.at[combo_v], rows_v, sem).wait()
        pltpu.sync_copy(rows_v, aggr_sh.at[dst_v], add=True)
        return carry

    lax.fori_loop(0, NCHUNK, chunk, 0)
    plsc.subcore_barrier()
    pltpu.sync_copy(aggr_sh.at[pl.ds(s * ZROWS, ZROWS)],
                    out_hbm.at[c, pl.ds(s * ZROWS, ZROWS)])


@functools.cache
def _make_edge_kernel():
    return functools.partial(
        pl.kernel,
        out_type=jax.ShapeDtypeStruct((NC, NPAD, H), jnp.float32),
        mesh=plsc.VectorSubcoreMesh(core_axis_name="c", subcore_axis_name="s"),
        compiler_params=pltpu.CompilerParams(needs_layout_passes=False),
        scratch_types=[
            pltpu.VMEM((N,), jnp.int32),
            pltpu.VMEM((CH,), jnp.int32),
            pltpu.VMEM((CH,), jnp.int32),
            pltpu.VMEM((CH,), jnp.int32),
            pltpu.VMEM((CH,), jnp.int32),
            pltpu.VMEM((CH, H), jnp.float32),
            pltpu.VMEM_SHARED((NPAD, H), jnp.float32),
            pltpu.SemaphoreType.DMA,
        ],
    )(_edge_body)


def _final_body(p_ref, x_ref, b_ref, ne, wih, whh, bih, bhh,
                riW1, rib1, riW2, rib2, rjW1, rjb1, rjW2, rjb2, out):
    i = pl.program_id(0)
    aggr = p_ref[0] + p_ref[1]
    xv = x_ref[...]
    onehot = (xv == lax.broadcasted_iota(jnp.int32, (BLK, NODE_VOCAB), 1)).astype(jnp.float32)
    h0 = jnp.dot(onehot, ne[...], preferred_element_type=jnp.float32)
    gi = lax.dot_general(aggr, wih[...], (((1,), (1,)), ((), ())),
                         preferred_element_type=jnp.float32) + bih[...]
    gh = lax.dot_general(h0, whh[...], (((1,), (1,)), ((), ())),
                         preferred_element_type=jnp.float32) + bhh[...]
    r = jax.nn.sigmoid(gi[:, :H] + gh[:, :H])
    z = jax.nn.sigmoid(gi[:, H:2 * H] + gh[:, H:2 * H])
    n = jnp.tanh(gi[:, 2 * H:] + r * gh[:, 2 * H:])
    h = (1.0 - z) * n + z * h0

    hi = jnp.maximum(jnp.dot(h, riW1[...], preferred_element_type=jnp.float32) + rib1[...], 0.0)
    i_score = jnp.dot(hi, riW2[...], preferred_element_type=jnp.float32) + rib2[...]
    hj = jnp.maximum(jnp.dot(h0, rjW1[...], preferred_element_type=jnp.float32) + rjb1[...], 0.0)
    j_score = jnp.dot(hj, rjW2[...], preferred_element_type=jnp.float32) + rjb2[...]

    bv = b_ref[...]
    probs = jax.nn.sigmoid(i_score + j_score)
    probs = jnp.where(bv >= 0, probs, 0.0)
    og = (bv == lax.broadcasted_iota(jnp.int32, (BLK, NUM_GRAPHS), 1)).astype(jnp.float32)
    contrib = lax.dot_general(probs, og, (((0,), (0,)), ((), ())),
                              preferred_element_type=jnp.float32)

    @pl.when(i == 0)
    def _():
        out[...] = jnp.zeros_like(out)

    out[...] += contrib.reshape(NUM_GRAPHS)


def _make_final(interpret=False):
    full = lambda shape: pl.BlockSpec(shape, lambda i: tuple(0 for _ in shape))
    return pl.pallas_call(
        _final_body,
        grid=(NBLK,),
        in_specs=[
            pl.BlockSpec((NC, BLK, H), lambda i: (0, i, 0)),
            pl.BlockSpec((BLK, 1), lambda i: (i, 0)),
            pl.BlockSpec((BLK, 1), lambda i: (i, 0)),
            full((NODE_VOCAB, H)),
            full((3 * H, H)),
            full((3 * H, H)),
            full((3 * H,)),
            full((3 * H,)),
            full((H, H)),
            full((H,)),
            full((H, 1)),
            full((1,)),
            full((H, H)),
            full((H,)),
            full((H, 1)),
            full((1,)),
        ],
        out_specs=pl.BlockSpec((NUM_GRAPHS,), lambda i: (0,)),
        out_shape=jax.ShapeDtypeStruct((NUM_GRAPHS,), jnp.float32),
        interpret=interpret,
    )


def kernel(x, edge_index, edge_attr, batch, node_emb, edge_emb, msg_W1, msg_b1,
           msg_W2, msg_b2, gru_Wih, gru_Whh, gru_bih, gru_bhh, ri_W1, ri_b1,
           ri_W2, ri_b2, rj_W1, rj_b1, rj_W2, rj_b2):
    xf = x.reshape(-1).astype(jnp.int32)
    src = edge_index[0].astype(jnp.int32)
    dst = edge_index[1].astype(jnp.int32)
    attr = edge_attr.reshape(-1).astype(jnp.int32)

    mtab = _make_table()(node_emb, edge_emb, msg_W1, msg_b1, msg_W2, msg_b2)
    zeros = jnp.zeros((ZROWS, H), jnp.float32)
    partials = _make_edge_kernel()(src, dst, attr, xf, mtab, zeros)

    x_pad = jnp.pad(xf, (0, NPAD - N)).reshape(NPAD, 1)
    b_pad = jnp.pad(batch.astype(jnp.int32), (0, NPAD - N),
                    constant_values=-1).reshape(NPAD, 1)
    return _make_final()(partials, x_pad, b_pad, node_emb, gru_Wih, gru_Whh,
                         gru_bih, gru_bhh, ri_W1, ri_b1, ri_W2, ri_b2,
                         rj_W1, rj_b1, rj_W2, rj_b2)


# speedup vs baseline: 17.4056x; 1.8910x over previous
import functools

import jax
import jax.numpy as jnp
from jax import lax
from jax.experimental import pallas as pl
from jax.experimental.pallas import tpu as pltpu
from jax.experimental.pallas import tpu_sc as plsc

N = 10000
E = 640000
H = 128
NODE_VOCAB = 128
EDGE_VOCAB = 16
NUM_GRAPHS = 256
NCOMBO = NODE_VOCAB * EDGE_VOCAB

NPAD = 10240
BLK = 1024
NBLK = NPAD // BLK

NC = 2
NS = 16
NW = NC * NS
CH = 128
NCHUNK = -(-E // (NW * CH))
EPT = NCHUNK * CH
EPAD = NW * EPT
DEPTH = 2
SAFE_ROW = NPAD - 1
ZROWS = NPAD // NS


def _table_body(ne, ee, w1, b1, w2, b2, out):
    pre_n = jnp.dot(ne[...], w1[:H, :], preferred_element_type=jnp.float32)
    pre_e = jnp.dot(ee[...], w1[H:, :], preferred_element_type=jnp.float32) + b1[...]
    row = lax.broadcasted_iota(jnp.int32, (NCOMBO, 1), 0)
    oa = (row // EDGE_VOCAB == lax.broadcasted_iota(jnp.int32, (NCOMBO, NODE_VOCAB), 1)).astype(jnp.float32)
    ob = (row % EDGE_VOCAB == lax.broadcasted_iota(jnp.int32, (NCOMBO, EDGE_VOCAB), 1)).astype(jnp.float32)
    z = jnp.dot(oa, pre_n, preferred_element_type=jnp.float32)
    z = z + jnp.dot(ob, pre_e, preferred_element_type=jnp.float32)
    out[...] = jnp.dot(jnp.maximum(z, 0.0), w2[...], preferred_element_type=jnp.float32) + b2[...]


def _make_table(interpret=False):
    return pl.pallas_call(
        _table_body,
        out_shape=jax.ShapeDtypeStruct((NCOMBO, H), jnp.float32),
        interpret=interpret,
    )


def _edge_body(src_hbm, dst_hbm, attr_hbm, x_hbm, mtab_hbm, zeros_hbm, out_hbm,
               x_v, src_v, dst_v, combo_v, ring_v, aggr_sh, lsem, gsem, ssem):
    c = lax.axis_index("c")
    s = lax.axis_index("s")
    w = c * NS + s

    pltpu.sync_copy(zeros_hbm, aggr_sh.at[pl.ds(s * ZROWS, ZROWS)])
    pltpu.sync_copy(x_hbm, x_v)
    plsc.subcore_barrier()

    def start_linear(j):
        pltpu.async_copy(src_hbm.at[w, j], src_v.at[lax.rem(j, 2)], lsem)
        pltpu.async_copy(dst_hbm.at[w, j], dst_v.at[lax.rem(j, 3)], lsem)
        pltpu.async_copy(attr_hbm.at[w, j], combo_v.at[lax.rem(j, 2)], lsem)

    def wait_linear(j):
        pltpu.make_async_copy(src_hbm.at[w, j], src_v.at[lax.rem(j, 2)],
                              lsem).wait()
        pltpu.make_async_copy(dst_hbm.at[w, j], dst_v.at[lax.rem(j, 3)],
                              lsem).wait()
        pltpu.make_async_copy(attr_hbm.at[w, j], combo_v.at[lax.rem(j, 2)],
                              lsem).wait()

    def compute_combo(jm):
        def c16(k, carry2):
            sl = pl.ds(k * 16, 16)
            xg = plsc.load_gather(x_v, [src_v[jm, sl]])
            combo_v[jm, sl] = xg * EDGE_VOCAB + combo_v[jm, sl]
            return carry2
        lax.fori_loop(0, CH // 16, c16, 0, unroll=True)

    def wait_gather(j):
        pltpu.make_async_copy(mtab_hbm.at[combo_v.at[lax.rem(j, 2)]],
                              ring_v.at[lax.rem(j, 2)], gsem).wait()

    def start_scatter(j):
        pltpu.async_copy(ring_v.at[lax.rem(j, 2)],
                         aggr_sh.at[dst_v.at[lax.rem(j, 3)]], ssem, add=True)

    def wait_scatter(j):
        pltpu.make_async_copy(ring_v.at[lax.rem(j, 2)],
                              aggr_sh.at[dst_v.at[lax.rem(j, 3)]],
                              ssem).wait()

    start_linear(0)

    def chunk(j, carry):
        @pl.when(j >= 1)
        def _():
            wait_gather(j - 1)
            start_scatter(j - 1)

        @pl.when(j >= 2)
        def _():
            wait_scatter(j - 2)

        wait_linear(j)

        @pl.when(j + 1 < NCHUNK)
        def _():
            start_linear(j + 1)

        compute_combo(lax.rem(j, 2))
        pltpu.async_copy(mtab_hbm.at[combo_v.at[lax.rem(j, 2)]],
                         ring_v.at[lax.rem(j, 2)], gsem)
        return carry

    lax.fori_loop(0, NCHUNK, chunk, 0)
    wait_gather(NCHUNK - 1)
    start_scatter(NCHUNK - 1)
    wait_scatter(NCHUNK - 2)
    wait_scatter(NCHUNK - 1)

    plsc.subcore_barrier()
    pltpu.sync_copy(aggr_sh.at[pl.ds(s * ZROWS, ZROWS)],
                    out_hbm.at[c, pl.ds(s * ZROWS, ZROWS)])


@functools.cache
def _make_edge_kernel():
    return functools.partial(
        pl.kernel,
        out_type=jax.ShapeDtypeStruct((NC, NPAD, H), jnp.float32),
        mesh=plsc.VectorSubcoreMesh(core_axis_name="c", subcore_axis_name="s"),
        compiler_params=pltpu.CompilerParams(needs_layout_passes=False),
        scratch_types=[
            pltpu.VMEM((N,), jnp.int32),
            pltpu.VMEM((2, CH), jnp.int32),
            pltpu.VMEM((3, CH), jnp.int32),
            pltpu.VMEM((2, CH), jnp.int32),
            pltpu.VMEM((DEPTH, CH, H), jnp.float32),
            pltpu.VMEM_SHARED((NPAD, H), jnp.float32),
            pltpu.SemaphoreType.DMA,
            pltpu.SemaphoreType.DMA,
            pltpu.SemaphoreType.DMA,
        ],
    )(_edge_body)


def _final_body(p_ref, x_ref, b_ref, ne, wih, whh, bih, bhh,
                riW1, rib1, riW2, rib2, rjW1, rjb1, rjW2, rjb2, out):
    i = pl.program_id(0)
    aggr = p_ref[0] + p_ref[1]
    xv = x_ref[...]
    onehot = (xv == lax.broadcasted_iota(jnp.int32, (BLK, NODE_VOCAB), 1)).astype(jnp.float32)
    h0 = jnp.dot(onehot, ne[...], preferred_element_type=jnp.float32)
    gi = lax.dot_general(aggr, wih[...], (((1,), (1,)), ((), ())),
                         preferred_element_type=jnp.float32) + bih[...]
    gh = lax.dot_general(h0, whh[...], (((1,), (1,)), ((), ())),
                         preferred_element_type=jnp.float32) + bhh[...]
    r = jax.nn.sigmoid(gi[:, :H] + gh[:, :H])
    z = jax.nn.sigmoid(gi[:, H:2 * H] + gh[:, H:2 * H])
    n = jnp.tanh(gi[:, 2 * H:] + r * gh[:, 2 * H:])
    h = (1.0 - z) * n + z * h0

    hi = jnp.maximum(jnp.dot(h, riW1[...], preferred_element_type=jnp.float32) + rib1[...], 0.0)
    i_score = jnp.dot(hi, riW2[...], preferred_element_type=jnp.float32) + rib2[...]
    hj = jnp.maximum(jnp.dot(h0, rjW1[...], preferred_element_type=jnp.float32) + rjb1[...], 0.0)
    j_score = jnp.dot(hj, rjW2[...], preferred_element_type=jnp.float32) + rjb2[...]

    bv = b_ref[...]
    probs = jax.nn.sigmoid(i_score + j_score)
    probs = jnp.where(bv >= 0, probs, 0.0)
    og = (bv == lax.broadcasted_iota(jnp.int32, (BLK, NUM_GRAPHS), 1)).astype(jnp.float32)
    contrib = lax.dot_general(probs, og, (((0,), (0,)), ((), ())),
                              preferred_element_type=jnp.float32)

    @pl.when(i == 0)
    def _():
        out[...] = jnp.zeros_like(out)

    out[...] += contrib.reshape(NUM_GRAPHS)


def _make_final(interpret=False):
    full = lambda shape: pl.BlockSpec(shape, lambda i: tuple(0 for _ in shape))
    return pl.pallas_call(
        _final_body,
        grid=(NBLK,),
        in_specs=[
            pl.BlockSpec((NC, BLK, H), lambda i: (0, i, 0)),
            pl.BlockSpec((BLK, 1), lambda i: (i, 0)),
            pl.BlockSpec((BLK, 1), lambda i: (i, 0)),
            full((NODE_VOCAB, H)),
            full((3 * H, H)),
            full((3 * H, H)),
            full((3 * H,)),
            full((3 * H,)),
            full((H, H)),
            full((H,)),
            full((H, 1)),
            full((1,)),
            full((H, H)),
            full((H,)),
            full((H, 1)),
            full((1,)),
        ],
        out_specs=pl.BlockSpec((NUM_GRAPHS,), lambda i: (0,)),
        out_shape=jax.ShapeDtypeStruct((NUM_GRAPHS,), jnp.float32),
        interpret=interpret,
    )


def kernel(x, edge_index, edge_attr, batch, node_emb, edge_emb, msg_W1, msg_b1,
           msg_W2, msg_b2, gru_Wih, gru_Whh, gru_bih, gru_bhh, ri_W1, ri_b1,
           ri_W2, ri_b2, rj_W1, rj_b1, rj_W2, rj_b2):
    xf = x.reshape(-1).astype(jnp.int32)
    pad = (0, EPAD - E)
    src = jnp.pad(edge_index[0].astype(jnp.int32), pad).reshape(NW, NCHUNK, CH)
    dst = jnp.pad(edge_index[1].astype(jnp.int32), pad,
                  constant_values=SAFE_ROW).reshape(NW, NCHUNK, CH)
    attr = jnp.pad(edge_attr.reshape(-1).astype(jnp.int32),
                   pad).reshape(NW, NCHUNK, CH)

    mtab = _make_table()(node_emb, edge_emb, msg_W1, msg_b1, msg_W2, msg_b2)
    zeros = jnp.zeros((ZROWS, H), jnp.float32)
    partials = _make_edge_kernel()(src, dst, attr, xf, mtab, zeros)

    x_pad = jnp.pad(xf, (0, NPAD - N)).reshape(NPAD, 1)
    b_pad = jnp.pad(batch.astype(jnp.int32), (0, NPAD - N),
                    constant_values=-1).reshape(NPAD, 1)
    return _make_final()(partials, x_pad, b_pad, node_emb, gru_Wih, gru_Whh,
                         gru_bih, gru_bhh, ri_W1, ri_b1, ri_W2, ri_b2,
                         rj_W1, rj_b1, rj_W2, rj_b2)


# speedup vs baseline: 20.8498x; 1.1979x over previous
import functools

import jax
import jax.numpy as jnp
from jax import lax
from jax.experimental import pallas as pl
from jax.experimental.pallas import tpu as pltpu
from jax.experimental.pallas import tpu_sc as plsc

N = 10000
E = 640000
H = 128
NODE_VOCAB = 128
EDGE_VOCAB = 16
NUM_GRAPHS = 256
NCOMBO = NODE_VOCAB * EDGE_VOCAB

NPAD = 10112
BLK = 1264
NBLK = NPAD // BLK

NC = 2
NS = 16
NW = NC * NS
CH = 96
NCHUNK = -(-E // (NW * CH))
EPT = NCHUNK * CH
EPAD = NW * EPT
RING_D = 3
SRC_D = 2
COMBO_D = 3
DST_D = 4
SAFE_ROW = NPAD - 1
ZROWS = NPAD // NS


def _table_body(ne, ee, w1, b1, w2, b2, out):
    pre_n = jnp.dot(ne[...], w1[:H, :], preferred_element_type=jnp.float32)
    pre_e = jnp.dot(ee[...], w1[H:, :], preferred_element_type=jnp.float32) + b1[...]
    row = lax.broadcasted_iota(jnp.int32, (NCOMBO, 1), 0)
    oa = (row // EDGE_VOCAB == lax.broadcasted_iota(jnp.int32, (NCOMBO, NODE_VOCAB), 1)).astype(jnp.float32)
    ob = (row % EDGE_VOCAB == lax.broadcasted_iota(jnp.int32, (NCOMBO, EDGE_VOCAB), 1)).astype(jnp.float32)
    z = jnp.dot(oa, pre_n, preferred_element_type=jnp.float32)
    z = z + jnp.dot(ob, pre_e, preferred_element_type=jnp.float32)
    out[...] = jnp.dot(jnp.maximum(z, 0.0), w2[...], preferred_element_type=jnp.float32) + b2[...]


def _make_table(interpret=False):
    return pl.pallas_call(
        _table_body,
        out_shape=jax.ShapeDtypeStruct((NCOMBO, H), jnp.float32),
        interpret=interpret,
    )


def _edge_body(src_hbm, dst_hbm, attr_hbm, x_hbm, mtab_hbm, zeros_hbm, out_hbm,
               x_v, src_v, dst_v, combo_v, ring_v, aggr_sh, lsem, gsem, ssem):
    c = lax.axis_index("c")
    s = lax.axis_index("s")
    w = c * NS + s

    pltpu.sync_copy(zeros_hbm, aggr_sh.at[pl.ds(s * ZROWS, ZROWS)])
    pltpu.sync_copy(x_hbm, x_v)
    plsc.subcore_barrier()

    def start_linear(j):
        pltpu.async_copy(src_hbm.at[w, j], src_v.at[lax.rem(j, SRC_D)], lsem)
        pltpu.async_copy(dst_hbm.at[w, j], dst_v.at[lax.rem(j, DST_D)], lsem)
        pltpu.async_copy(attr_hbm.at[w, j], combo_v.at[lax.rem(j, COMBO_D)],
                         lsem)

    def wait_linear(j):
        pltpu.make_async_copy(src_hbm.at[w, j], src_v.at[lax.rem(j, SRC_D)],
                              lsem).wait()
        pltpu.make_async_copy(dst_hbm.at[w, j], dst_v.at[lax.rem(j, DST_D)],
                              lsem).wait()
        pltpu.make_async_copy(attr_hbm.at[w, j],
                              combo_v.at[lax.rem(j, COMBO_D)], lsem).wait()

    def compute_combo(j):
        js = lax.rem(j, SRC_D)
        jc = lax.rem(j, COMBO_D)

        def c16(k, carry2):
            sl = pl.ds(k * 16, 16)
            xg = plsc.load_gather(x_v, [src_v[js, sl]])
            combo_v[jc, sl] = xg * EDGE_VOCAB + combo_v[jc, sl]
            return carry2
        lax.fori_loop(0, CH // 16, c16, 0, unroll=True)

    def start_gather(j):
        pltpu.async_copy(mtab_hbm.at[combo_v.at[lax.rem(j, COMBO_D)]],
                         ring_v.at[lax.rem(j, RING_D)], gsem)

    def wait_gather(j):
        pltpu.make_async_copy(mtab_hbm.at[combo_v.at[lax.rem(j, COMBO_D)]],
                              ring_v.at[lax.rem(j, RING_D)], gsem).wait()

    def start_scatter(j):
        pltpu.async_copy(ring_v.at[lax.rem(j, RING_D)],
                         aggr_sh.at[dst_v.at[lax.rem(j, DST_D)]], ssem,
                         add=True)

    def wait_scatter(j):
        pltpu.make_async_copy(ring_v.at[lax.rem(j, RING_D)],
                              aggr_sh.at[dst_v.at[lax.rem(j, DST_D)]],
                              ssem).wait()

    start_linear(0)
    start_linear(1)
    wait_linear(0)
    compute_combo(0)
    start_gather(0)

    def chunk(j, carry):
        @pl.when(j >= 1)
        def _():
            wait_gather(j - 1)
            start_scatter(j - 1)

        @pl.when(j >= 2)
        def _():
            wait_scatter(j - 2)

        @pl.when(j + 1 < NCHUNK)
        def _():
            wait_linear(j + 1)

        @pl.when(j + 2 < NCHUNK)
        def _():
            start_linear(j + 2)

        @pl.when(j + 1 < NCHUNK)
        def _():
            compute_combo(j + 1)
            start_gather(j + 1)

        return carry

    lax.fori_loop(0, NCHUNK, chunk, 0)
    wait_gather(NCHUNK - 1)
    start_scatter(NCHUNK - 1)
    wait_scatter(NCHUNK - 2)
    wait_scatter(NCHUNK - 1)

    plsc.subcore_barrier()
    pltpu.sync_copy(aggr_sh.at[pl.ds(s * ZROWS, ZROWS)],
                    out_hbm.at[c, pl.ds(s * ZROWS, ZROWS)])


@functools.cache
def _make_edge_kernel():
    return functools.partial(
        pl.kernel,
        out_type=jax.ShapeDtypeStruct((NC, NPAD, H), jnp.float32),
        mesh=plsc.VectorSubcoreMesh(core_axis_name="c", subcore_axis_name="s"),
        compiler_params=pltpu.CompilerParams(needs_layout_passes=False),
        scratch_types=[
            pltpu.VMEM((N,), jnp.int32),
            pltpu.VMEM((SRC_D, CH), jnp.int32),
            pltpu.VMEM((DST_D, CH), jnp.int32),
            pltpu.VMEM((COMBO_D, CH), jnp.int32),
            pltpu.VMEM((RING_D, CH, H), jnp.float32),
            pltpu.VMEM_SHARED((NPAD, H), jnp.float32),
            pltpu.SemaphoreType.DMA,
            pltpu.SemaphoreType.DMA,
            pltpu.SemaphoreType.DMA,
        ],
    )(_edge_body)


def _final_body(p_ref, x_ref, b_ref, ne, wih, whh, bih, bhh,
                riW1, rib1, riW2, rib2, rjW1, rjb1, rjW2, rjb2, out):
    i = pl.program_id(0)
    aggr = p_ref[0] + p_ref[1]
    xv = x_ref[...]
    onehot = (xv == lax.broadcasted_iota(jnp.int32, (BLK, NODE_VOCAB), 1)).astype(jnp.float32)
    h0 = jnp.dot(onehot, ne[...], preferred_element_type=jnp.float32)
    gi = lax.dot_general(aggr, wih[...], (((1,), (1,)), ((), ())),
                         preferred_element_type=jnp.float32) + bih[...]
    gh = lax.dot_general(h0, whh[...], (((1,), (1,)), ((), ())),
                         preferred_element_type=jnp.float32) + bhh[...]
    r = jax.nn.sigmoid(gi[:, :H] + gh[:, :H])
    z = jax.nn.sigmoid(gi[:, H:2 * H] + gh[:, H:2 * H])
    n = jnp.tanh(gi[:, 2 * H:] + r * gh[:, 2 * H:])
    h = (1.0 - z) * n + z * h0

    hi = jnp.maximum(jnp.dot(h, riW1[...], preferred_element_type=jnp.float32) + rib1[...], 0.0)
    i_score = jnp.dot(hi, riW2[...], preferred_element_type=jnp.float32) + rib2[...]
    hj = jnp.maximum(jnp.dot(h0, rjW1[...], preferred_element_type=jnp.float32) + rjb1[...], 0.0)
    j_score = jnp.dot(hj, rjW2[...], preferred_element_type=jnp.float32) + rjb2[...]

    bv = b_ref[...]
    probs = jax.nn.sigmoid(i_score + j_score)
    probs = jnp.where(bv >= 0, probs, 0.0)
    og = (bv == lax.broadcasted_iota(jnp.int32, (BLK, NUM_GRAPHS), 1)).astype(jnp.float32)
    contrib = lax.dot_general(probs, og, (((0,), (0,)), ((), ())),
                              preferred_element_type=jnp.float32)

    @pl.when(i == 0)
    def _():
        out[...] = jnp.zeros_like(out)

    out[...] += contrib.reshape(NUM_GRAPHS)


def _make_final(interpret=False):
    full = lambda shape: pl.BlockSpec(shape, lambda i: tuple(0 for _ in shape))
    return pl.pallas_call(
        _final_body,
        grid=(NBLK,),
        in_specs=[
            pl.BlockSpec((NC, BLK, H), lambda i: (0, i, 0)),
            pl.BlockSpec((BLK, 1), lambda i: (i, 0)),
            pl.BlockSpec((BLK, 1), lambda i: (i, 0)),
            full((NODE_VOCAB, H)),
            full((3 * H, H)),
            full((3 * H, H)),
            full((3 * H,)),
            full((3 * H,)),
            full((H, H)),
            full((H,)),
            full((H, 1)),
            full((1,)),
            full((H, H)),
            full((H,)),
            full((H, 1)),
            full((1,)),
        ],
        out_specs=pl.BlockSpec((NUM_GRAPHS,), lambda i: (0,)),
        out_shape=jax.ShapeDtypeStruct((NUM_GRAPHS,), jnp.float32),
        interpret=interpret,
    )


def kernel(x, edge_index, edge_attr, batch, node_emb, edge_emb, msg_W1, msg_b1,
           msg_W2, msg_b2, gru_Wih, gru_Whh, gru_bih, gru_bhh, ri_W1, ri_b1,
           ri_W2, ri_b2, rj_W1, rj_b1, rj_W2, rj_b2):
    xf = x.reshape(-1).astype(jnp.int32)
    pad = (0, EPAD - E)
    src = jnp.pad(edge_index[0].astype(jnp.int32), pad).reshape(NW, NCHUNK, CH)
    dst = jnp.pad(edge_index[1].astype(jnp.int32), pad,
                  constant_values=SAFE_ROW).reshape(NW, NCHUNK, CH)
    attr = jnp.pad(edge_attr.reshape(-1).astype(jnp.int32),
                   pad).reshape(NW, NCHUNK, CH)

    mtab = _make_table()(node_emb, edge_emb, msg_W1, msg_b1, msg_W2, msg_b2)
    zeros = jnp.zeros((ZROWS, H), jnp.float32)
    partials = _make_edge_kernel()(src, dst, attr, xf, mtab, zeros)

    x_pad = jnp.pad(xf, (0, NPAD - N)).reshape(NPAD, 1)
    b_pad = jnp.pad(batch.astype(jnp.int32), (0, NPAD - N),
                    constant_values=-1).reshape(NPAD, 1)
    return _make_final()(partials, x_pad, b_pad, node_emb, gru_Wih, gru_Whh,
                         gru_bih, gru_bhh, ri_W1, ri_b1, ri_W2, ri_b2,
                         rj_W1, rj_b1, rj_W2, rj_b2)


# speedup vs baseline: 22.0794x; 1.0590x over previous
import functools

import jax
import jax.numpy as jnp
from jax import lax
from jax.experimental import pallas as pl
from jax.experimental.pallas import tpu as pltpu
from jax.experimental.pallas import tpu_sc as plsc

N = 10000
E = 640000
H = 128
NODE_VOCAB = 128
EDGE_VOCAB = 16
NUM_GRAPHS = 256
NCOMBO = NODE_VOCAB * EDGE_VOCAB

NPAD = 10112
BLK = 1264
XP = N // 4
NBLK = NPAD // BLK

NC = 2
NS = 16
NW = NC * NS
CH = 112
NCHUNK = -(-E // (NW * CH))
EPT = NCHUNK * CH
EPAD = NW * EPT
RING_D = 3
SRC_D = 2
COMBO_D = 3
DST_D = 4
SAFE_ROW = NPAD - 1
ZROWS = NPAD // NS


def _table_body(ne, ee, w1, b1, w2, b2, out):
    pre_n = jnp.dot(ne[...], w1[:H, :], preferred_element_type=jnp.float32)
    pre_e = jnp.dot(ee[...], w1[H:, :], preferred_element_type=jnp.float32) + b1[...]
    row = lax.broadcasted_iota(jnp.int32, (NCOMBO, 1), 0)
    oa = (row // EDGE_VOCAB == lax.broadcasted_iota(jnp.int32, (NCOMBO, NODE_VOCAB), 1)).astype(jnp.float32)
    ob = (row % EDGE_VOCAB == lax.broadcasted_iota(jnp.int32, (NCOMBO, EDGE_VOCAB), 1)).astype(jnp.float32)
    z = jnp.dot(oa, pre_n, preferred_element_type=jnp.float32)
    z = z + jnp.dot(ob, pre_e, preferred_element_type=jnp.float32)
    out[...] = jnp.dot(jnp.maximum(z, 0.0), w2[...], preferred_element_type=jnp.float32) + b2[...]


def _make_table(interpret=False):
    return pl.pallas_call(
        _table_body,
        out_shape=jax.ShapeDtypeStruct((NCOMBO, H), jnp.float32),
        interpret=interpret,
    )


def _edge_body(src_hbm, dst_hbm, attr_hbm, x_hbm, mtab_hbm, zeros_hbm, out_hbm,
               x_v, src_v, dst_v, combo_v, ring_v, aggr_sh, lsem, gsem, ssem):
    c = lax.axis_index("c")
    s = lax.axis_index("s")
    w = c * NS + s

    pltpu.sync_copy(zeros_hbm, aggr_sh.at[pl.ds(s * ZROWS, ZROWS)])
    pltpu.sync_copy(x_hbm, x_v)
    plsc.subcore_barrier()

    def start_linear(j):
        pltpu.async_copy(src_hbm.at[w, j], src_v.at[lax.rem(j, SRC_D)], lsem)
        pltpu.async_copy(dst_hbm.at[w, j], dst_v.at[lax.rem(j, DST_D)], lsem)
        pltpu.async_copy(attr_hbm.at[w, j], combo_v.at[lax.rem(j, COMBO_D)],
                         lsem)

    def wait_linear(j):
        pltpu.make_async_copy(src_hbm.at[w, j], src_v.at[lax.rem(j, SRC_D)],
                              lsem).wait()
        pltpu.make_async_copy(dst_hbm.at[w, j], dst_v.at[lax.rem(j, DST_D)],
                              lsem).wait()
        pltpu.make_async_copy(attr_hbm.at[w, j],
                              combo_v.at[lax.rem(j, COMBO_D)], lsem).wait()

    def compute_combo(j):
        js = lax.rem(j, SRC_D)
        jc = lax.rem(j, COMBO_D)

        def c16(k, carry2):
            sl = pl.ds(k * 16, 16)
            srcv = src_v[js, sl]
            word = plsc.load_gather(x_v, [srcv >> 2])
            xg = (word >> ((srcv & 3) << 3)) & 0xFF
            combo_v[jc, sl] = xg * EDGE_VOCAB + combo_v[jc, sl]
            return carry2
        lax.fori_loop(0, CH // 16, c16, 0, unroll=True)

    def start_gather(j):
        pltpu.async_copy(mtab_hbm.at[combo_v.at[lax.rem(j, COMBO_D)]],
                         ring_v.at[lax.rem(j, RING_D)], gsem)

    def wait_gather(j):
        pltpu.make_async_copy(mtab_hbm.at[combo_v.at[lax.rem(j, COMBO_D)]],
                              ring_v.at[lax.rem(j, RING_D)], gsem).wait()

    def start_scatter(j):
        pltpu.async_copy(ring_v.at[lax.rem(j, RING_D)],
                         aggr_sh.at[dst_v.at[lax.rem(j, DST_D)]], ssem,
                         add=True)

    def wait_scatter(j):
        pltpu.make_async_copy(ring_v.at[lax.rem(j, RING_D)],
                              aggr_sh.at[dst_v.at[lax.rem(j, DST_D)]],
                              ssem).wait()

    start_linear(0)
    start_linear(1)
    wait_linear(0)
    compute_combo(0)
    start_gather(0)

    def chunk(j, carry):
        @pl.when(j >= 1)
        def _():
            wait_gather(j - 1)
            start_scatter(j - 1)

        @pl.when(j >= 2)
        def _():
            wait_scatter(j - 2)

        @pl.when(j + 1 < NCHUNK)
        def _():
            wait_linear(j + 1)

        @pl.when(j + 2 < NCHUNK)
        def _():
            start_linear(j + 2)

        @pl.when(j + 1 < NCHUNK)
        def _():
            compute_combo(j + 1)
            start_gather(j + 1)

        return carry

    lax.fori_loop(0, NCHUNK, chunk, 0)
    wait_gather(NCHUNK - 1)
    start_scatter(NCHUNK - 1)
    wait_scatter(NCHUNK - 2)
    wait_scatter(NCHUNK - 1)

    plsc.subcore_barrier()
    pltpu.sync_copy(aggr_sh.at[pl.ds(s * ZROWS, ZROWS)],
                    out_hbm.at[c, pl.ds(s * ZROWS, ZROWS)])


@functools.cache
def _make_edge_kernel():
    return functools.partial(
        pl.kernel,
        out_type=jax.ShapeDtypeStruct((NC, NPAD, H), jnp.float32),
        mesh=plsc.VectorSubcoreMesh(core_axis_name="c", subcore_axis_name="s"),
        compiler_params=pltpu.CompilerParams(needs_layout_passes=False),
        scratch_types=[
            pltpu.VMEM((XP,), jnp.int32),
            pltpu.VMEM((SRC_D, CH), jnp.int32),
            pltpu.VMEM((DST_D, CH), jnp.int32),
            pltpu.VMEM((COMBO_D, CH), jnp.int32),
            pltpu.VMEM((RING_D, CH, H), jnp.float32),
            pltpu.VMEM_SHARED((NPAD, H), jnp.float32),
            pltpu.SemaphoreType.DMA,
            pltpu.SemaphoreType.DMA,
            pltpu.SemaphoreType.DMA,
        ],
    )(_edge_body)


def _final_body(p_ref, x_ref, b_ref, ne, wih, whh, bih, bhh,
                riW1, rib1, riW2, rib2, rjW1, rjb1, rjW2, rjb2, out):
    i = pl.program_id(0)
    aggr = p_ref[0] + p_ref[1]
    xv = x_ref[...]
    onehot = (xv == lax.broadcasted_iota(jnp.int32, (BLK, NODE_VOCAB), 1)).astype(jnp.float32)
    h0 = jnp.dot(onehot, ne[...], preferred_element_type=jnp.float32)
    gi = lax.dot_general(aggr, wih[...], (((1,), (1,)), ((), ())),
                         preferred_element_type=jnp.float32) + bih[...]
    gh = lax.dot_general(h0, whh[...], (((1,), (1,)), ((), ())),
                         preferred_element_type=jnp.float32) + bhh[...]
    r = jax.nn.sigmoid(gi[:, :H] + gh[:, :H])
    z = jax.nn.sigmoid(gi[:, H:2 * H] + gh[:, H:2 * H])
    n = jnp.tanh(gi[:, 2 * H:] + r * gh[:, 2 * H:])
    h = (1.0 - z) * n + z * h0

    hi = jnp.maximum(jnp.dot(h, riW1[...], preferred_element_type=jnp.float32) + rib1[...], 0.0)
    i_score = jnp.dot(hi, riW2[...], preferred_element_type=jnp.float32) + rib2[...]
    hj = jnp.maximum(jnp.dot(h0, rjW1[...], preferred_element_type=jnp.float32) + rjb1[...], 0.0)
    j_score = jnp.dot(hj, rjW2[...], preferred_element_type=jnp.float32) + rjb2[...]

    bv = b_ref[...]
    probs = jax.nn.sigmoid(i_score + j_score)
    probs = jnp.where(bv >= 0, probs, 0.0)
    og = (bv == lax.broadcasted_iota(jnp.int32, (BLK, NUM_GRAPHS), 1)).astype(jnp.float32)
    contrib = lax.dot_general(probs, og, (((0,), (0,)), ((), ())),
                              preferred_element_type=jnp.float32)

    @pl.when(i == 0)
    def _():
        out[...] = jnp.zeros_like(out)

    out[...] += contrib.reshape(NUM_GRAPHS)


def _make_final(interpret=False):
    full = lambda shape: pl.BlockSpec(shape, lambda i: tuple(0 for _ in shape))
    return pl.pallas_call(
        _final_body,
        grid=(NBLK,),
        in_specs=[
            pl.BlockSpec((NC, BLK, H), lambda i: (0, i, 0)),
            pl.BlockSpec((BLK, 1), lambda i: (i, 0)),
            pl.BlockSpec((BLK, 1), lambda i: (i, 0)),
            full((NODE_VOCAB, H)),
            full((3 * H, H)),
            full((3 * H, H)),
            full((3 * H,)),
            full((3 * H,)),
            full((H, H)),
            full((H,)),
            full((H, 1)),
            full((1,)),
            full((H, H)),
            full((H,)),
            full((H, 1)),
            full((1,)),
        ],
        out_specs=pl.BlockSpec((NUM_GRAPHS,), lambda i: (0,)),
        out_shape=jax.ShapeDtypeStruct((NUM_GRAPHS,), jnp.float32),
        interpret=interpret,
    )


def kernel(x, edge_index, edge_attr, batch, node_emb, edge_emb, msg_W1, msg_b1,
           msg_W2, msg_b2, gru_Wih, gru_Whh, gru_bih, gru_bhh, ri_W1, ri_b1,
           ri_W2, ri_b2, rj_W1, rj_b1, rj_W2, rj_b2):
    xf = x.reshape(-1).astype(jnp.int32)
    pad = (0, EPAD - E)
    src = jnp.pad(edge_index[0].astype(jnp.int32), pad).reshape(NW, NCHUNK, CH)
    dst = jnp.pad(edge_index[1].astype(jnp.int32), pad,
                  constant_values=SAFE_ROW).reshape(NW, NCHUNK, CH)
    attr = jnp.pad(edge_attr.reshape(-1).astype(jnp.int32),
                   pad).reshape(NW, NCHUNK, CH)

    xr = xf.reshape(XP, 4)
    xp = xr[:, 0] | (xr[:, 1] << 8) | (xr[:, 2] << 16) | (xr[:, 3] << 24)

    mtab = _make_table()(node_emb, edge_emb, msg_W1, msg_b1, msg_W2, msg_b2)
    zeros = jnp.zeros((ZROWS, H), jnp.float32)
    partials = _make_edge_kernel()(src, dst, attr, xp, mtab, zeros)

    x_pad = jnp.pad(xf, (0, NPAD - N)).reshape(NPAD, 1)
    b_pad = jnp.pad(batch.astype(jnp.int32), (0, NPAD - N),
                    constant_values=-1).reshape(NPAD, 1)
    return _make_final()(partials, x_pad, b_pad, node_emb, gru_Wih, gru_Whh,
                         gru_bih, gru_bhh, ri_W1, ri_b1, ri_W2, ri_b2,
                         rj_W1, rj_b1, rj_W2, rj_b2)
